# baseline (speedup 1.0000x reference)
import functools

import jax
import jax.numpy as jnp
from jax import lax
from jax.experimental import pallas as pl
from jax.experimental.pallas import tpu as pltpu
from jax.experimental.pallas import tpu_sc as plsc

EMB = 128
HID = 128

_NC, _NS = 2, 16
_NW = _NC * _NS
_CH = 50
_NBUF = 4
_S = 2
_BB = 16


_IW = 100
_NG = 2
_CHUNK = _IW * _NG


@functools.partial(jax.jit, static_argnums=(2, 3))
def _gather_raw(table, idx3, n_rows, n_ih):
    per_w = n_ih * _IW
    n_chunks = per_w // _CHUNK
    n_groups = n_chunks // _NBUF
    mesh = plsc.VectorSubcoreMesh(core_axis_name="c", subcore_axis_name="s")

    @functools.partial(
        pl.kernel,
        mesh=mesh,
        out_type=jax.ShapeDtypeStruct((n_rows, EMB), jnp.float32),
        scratch_types=[
            pltpu.VMEM((n_ih, _IW), jnp.int32),
        ] + [pltpu.VMEM((_CHUNK, EMB), jnp.float32) for _ in range(_NBUF)]
          + [pltpu.SemaphoreType.DMA for _ in range(2 * _NBUF)],
    )
    def gather_k(t_hbm, idx_hbm, out_hbm, idx_v,
                 b0, b1, b2, b3, g0, g1, g2, g3, o0, o1, o2, o3):
        bufs = (b0, b1, b2, b3)
        gsem = (g0, g1, g2, g3)
        osem = (o0, o1, o2, o3)
        wid = lax.axis_index("s") * _NC + lax.axis_index("c")
        base = wid * per_w
        pltpu.sync_copy(idx_hbm.at[wid], idx_v)

        def fire(j, b):
            for k in range(_NG):
                pltpu.async_copy(
                    t_hbm.at[idx_v.at[j * _NG + k]],
                    bufs[b].at[pl.ds(k * _IW, _IW)], gsem[b])

        def wait_fire(j, b):
            for k in range(_NG):
                pltpu.make_async_copy(
                    t_hbm.at[idx_v.at[j * _NG + k]],
                    bufs[b].at[pl.ds(k * _IW, _IW)], gsem[b]).wait()

        for b in range(_NBUF):
            fire(b, b)

        def group(g, carry):
            j0 = g * _NBUF
            for b in range(_NBUF):
                j = j0 + b
                dst = out_hbm.at[pl.ds(base + j * _CHUNK, _CHUNK)]
                wait_fire(j, b)
                pltpu.async_copy(bufs[b], dst, osem[b])

                @pl.when(g < n_groups - 1)
                def _():
                    pltpu.make_async_copy(bufs[b], dst, osem[b]).wait()
                    fire(j + _NBUF, b)
            return carry

        lax.fori_loop(0, n_groups, group, 0)
        last = (n_groups - 1) * _NBUF
        for b in range(_NBUF):
            j = last + b
            pltpu.make_async_copy(
                bufs[b], out_hbm.at[pl.ds(base + j * _CHUNK, _CHUNK)],
                osem[b]).wait()

    return gather_k(table, idx3)


def _proj_body(g_ref, w_ref, o_ref):
    w = w_ref[...]
    for bb in range(_BB):
        o_ref[bb, :, :] = lax.dot_general(
            g_ref[pl.ds(bb * _CH, _CH), :], w, (((1,), (1,)), ((), ())),
            preferred_element_type=jnp.float32)


def _proj_body_alias(g_ref, w_ref, prev_ref, o_ref):
    _proj_body(g_ref, w_ref, o_ref)


def _project_slice(g, nslice, W, batch, seq, b0, prev):
    grid = nslice // _BB
    in_specs = [
        pl.BlockSpec((_BB * seq, EMB), lambda i: (i, 0)),
        pl.BlockSpec((HID, EMB), lambda i: (0, 0)),
    ]
    args = [g, W]
    kwargs = {}
    body = _proj_body
    if prev is not None:
        in_specs.append(pl.BlockSpec(memory_space=pltpu.MemorySpace.HBM))
        args.append(prev)
        kwargs["input_output_aliases"] = {2: 0}
        body = _proj_body_alias
    blk0 = b0 // _BB
    return pl.pallas_call(
        body,
        grid=(grid,),
        in_specs=in_specs,
        out_specs=pl.BlockSpec((_BB, seq, HID), lambda i: (blk0 + i, 0, 0)),
        out_shape=jax.ShapeDtypeStruct((batch, seq, HID), jnp.float32),
        **kwargs,
    )(*args)


def kernel(prem, hypo, embed_table, W):
    B, L = prem.shape
    nslice = B // _S
    n_rows = nslice * L
    n_ih = n_rows // (_NW * _IW)

    gathered = []
    for idx in (prem, hypo):
        for s in range(_S):
            sl = lax.slice_in_dim(idx, s * nslice, (s + 1) * nslice)
            g = _gather_raw(embed_table, sl.reshape(_NW, n_ih, _IW),
                            n_rows, n_ih)
            gathered.append(g)

    outs = []
    for t in range(2):
        prev = None
        for s in range(_S):
            g = gathered[t * _S + s]
            prev = _project_slice(g, nslice, W, B, L, s * nslice, prev)
        outs.append(prev)
    return (outs[0], outs[1])


# speedup vs baseline: 1.0182x; 1.0182x over previous
import functools

import jax
import jax.numpy as jnp
from jax import lax
from jax.experimental import pallas as pl
from jax.experimental.pallas import tpu as pltpu
from jax.experimental.pallas import tpu_sc as plsc

EMB = 128
HID = 128

_NC, _NS = 2, 16
_NW = _NC * _NS
_LP = 56
_PB = 2
_NBUF = 4
_BB = 16


@functools.partial(jax.jit, static_argnums=(2, 3))
def _gather_raw(table, idx3, n_seq, seq):
    n_ih = n_seq // _NW
    chunk_rows = _PB * _LP
    n_chunks = n_ih // _PB
    n_groups = n_chunks // _NBUF
    mesh = plsc.VectorSubcoreMesh(core_axis_name="c", subcore_axis_name="s")

    @functools.partial(
        pl.kernel,
        mesh=mesh,
        out_type=jax.ShapeDtypeStruct((n_seq * _LP, EMB), jnp.float32),
        scratch_types=[
            pltpu.VMEM((n_ih, seq), jnp.int32),
        ] + [pltpu.VMEM((chunk_rows, EMB), jnp.float32) for _ in range(_NBUF)]
          + [pltpu.SemaphoreType.DMA for _ in range(2 * _NBUF)],
    )
    def gather_k(t_hbm, idx_hbm, out_hbm, idx_v,
                 b0, b1, b2, b3, g0, g1, g2, g3, o0, o1, o2, o3):
        bufs = (b0, b1, b2, b3)
        gsem = (g0, g1, g2, g3)
        osem = (o0, o1, o2, o3)
        wid = lax.axis_index("s") * _NC + lax.axis_index("c")
        base = wid * n_ih * _LP
        pltpu.sync_copy(idx_hbm.at[wid], idx_v)

        def fire(j, b):
            for k in range(_PB):
                pltpu.async_copy(
                    t_hbm.at[idx_v.at[j * _PB + k]],
                    bufs[b].at[pl.ds(k * _LP, seq)], gsem[b])

        def wait_fire(j, b):
            for k in range(_PB):
                pltpu.make_async_copy(
                    t_hbm.at[idx_v.at[j * _PB + k]],
                    bufs[b].at[pl.ds(k * _LP, seq)], gsem[b]).wait()

        for b in range(_NBUF):
            fire(b, b)

        def group(g, carry):
            j0 = g * _NBUF
            for b in range(_NBUF):
                j = j0 + b
                dst = out_hbm.at[pl.ds(base + j * chunk_rows, chunk_rows)]
                wait_fire(j, b)
                pltpu.async_copy(bufs[b], dst, osem[b])

                @pl.when(g < n_groups - 1)
                def _():
                    pltpu.make_async_copy(bufs[b], dst, osem[b]).wait()
                    fire(j + _NBUF, b)
            return carry

        lax.fori_loop(0, n_groups, group, 0)
        last = (n_groups - 1) * _NBUF
        for b in range(_NBUF):
            j = last + b
            pltpu.make_async_copy(
                bufs[b],
                out_hbm.at[pl.ds(base + j * chunk_rows, chunk_rows)],
                osem[b]).wait()

    return gather_k(table, idx3)


def _proj_body(g_ref, w_ref, o_ref):
    rows = lax.dot_general(
        g_ref[...], w_ref[...], (((1,), (1,)), ((), ())),
        preferred_element_type=jnp.float32)
    for bb in range(_BB):
        o_ref[bb, :, :] = rows[bb * _LP:bb * _LP + o_ref.shape[1], :]


def _project(g, W, batch, seq):
    grid = batch // _BB
    return pl.pallas_call(
        _proj_body,
        grid=(grid,),
        in_specs=[
            pl.BlockSpec((_BB * _LP, EMB), lambda i: (i, 0)),
            pl.BlockSpec((HID, EMB), lambda i: (0, 0)),
        ],
        out_specs=pl.BlockSpec((_BB, seq, HID), lambda i: (i, 0, 0)),
        out_shape=jax.ShapeDtypeStruct((batch, seq, HID), jnp.float32),
    )(g, W)


def kernel(prem, hypo, embed_table, W):
    B, L = prem.shape
    n_ih = B // _NW

    gathered = [
        _gather_raw(embed_table, idx.reshape(_NW, n_ih, L), B, L)
        for idx in (prem, hypo)
    ]
    outs = [_project(g, W, B, L) for g in gathered]
    return (outs[0], outs[1])


# speedup vs baseline: 2.0347x; 1.9983x over previous
import functools

import jax
import jax.numpy as jnp
from jax import lax
from jax.experimental import pallas as pl
from jax.experimental.pallas import tpu as pltpu
from jax.experimental.pallas import tpu_sc as plsc

EMB = 128
HID = 128

_NC, _NS = 2, 16
_NW = _NC * _NS
_NBUF = 4


_PROJ_BLOCK = 10000


def _proj_body(t_ref, w_ref, o_ref):
    i = pl.program_id(0)
    blk = t_ref[...]
    rows = lax.broadcasted_iota(jnp.int32, blk.shape, 0) + i * _PROJ_BLOCK
    blk = jnp.where(rows == 1, 0.0, blk)
    o_ref[...] = lax.dot_general(
        blk, w_ref[...], (((1,), (1,)), ((), ())),
        preferred_element_type=jnp.float32)


def _project_table(table, W):
    vocab = table.shape[0]
    grid = vocab // _PROJ_BLOCK
    return pl.pallas_call(
        _proj_body,
        grid=(grid,),
        in_specs=[
            pl.BlockSpec((_PROJ_BLOCK, EMB), lambda i: (i, 0)),
            pl.BlockSpec((HID, EMB), lambda i: (0, 0)),
        ],
        out_specs=pl.BlockSpec((_PROJ_BLOCK, HID), lambda i: (i, 0)),
        out_shape=jax.ShapeDtypeStruct((vocab, HID), jnp.float32),
    )(table, W)


@functools.partial(jax.jit, static_argnums=(2, 3))
def _gather_rows(p, idx3, batch, seq):
    n_ch = batch // _NW
    n_groups = n_ch // _NBUF
    mesh = plsc.VectorSubcoreMesh(core_axis_name="c", subcore_axis_name="s")

    @functools.partial(
        pl.kernel,
        mesh=mesh,
        out_type=jax.ShapeDtypeStruct((batch, seq, HID), jnp.float32),
        scratch_types=[
            pltpu.VMEM((n_ch, seq), jnp.int32),
        ] + [pltpu.VMEM((seq, HID), jnp.float32) for _ in range(_NBUF)]
          + [pltpu.SemaphoreType.DMA for _ in range(2 * _NBUF)],
    )
    def gather_k(p_hbm, idx_hbm, out_hbm, idx_v,
                 b0, b1, b2, b3, g0, g1, g2, g3, o0, o1, o2, o3):
        bufs = (b0, b1, b2, b3)
        gsem = (g0, g1, g2, g3)
        osem = (o0, o1, o2, o3)
        wid = lax.axis_index("s") * _NC + lax.axis_index("c")
        base = wid * n_ch
        pltpu.sync_copy(idx_hbm.at[wid], idx_v)
        for b in range(_NBUF):
            pltpu.async_copy(p_hbm.at[idx_v.at[b]], bufs[b], gsem[b])

        def group(g, carry):
            j0 = g * _NBUF
            for b in range(_NBUF):
                j = j0 + b
                dst = out_hbm.at[base + j]
                pltpu.make_async_copy(
                    p_hbm.at[idx_v.at[j]], bufs[b], gsem[b]).wait()
                pltpu.async_copy(bufs[b], dst, osem[b])

                @pl.when(g < n_groups - 1)
                def _():
                    pltpu.make_async_copy(bufs[b], dst, osem[b]).wait()
                    pltpu.async_copy(
                        p_hbm.at[idx_v.at[j + _NBUF]], bufs[b], gsem[b])
            return carry

        lax.fori_loop(0, n_groups, group, 0)
        last = (n_groups - 1) * _NBUF
        for b in range(_NBUF):
            j = last + b
            pltpu.make_async_copy(
                bufs[b], out_hbm.at[base + j], osem[b]).wait()

    return gather_k(p, idx3)


def kernel(prem, hypo, embed_table, W):
    B, L = prem.shape
    n_ch = B // _NW

    P = _project_table(embed_table, W)
    outs = [
        _gather_rows(P, idx.reshape(_NW, n_ch, L), B, L)
        for idx in (prem, hypo)
    ]
    return (outs[0], outs[1])
